# baseline (speedup 1.0000x reference)
import functools

import jax
import jax.numpy as jnp
from jax import lax
from jax.experimental import pallas as pl
from jax.experimental.pallas import tpu as pltpu
from jax.experimental.pallas import tpu_sc as plsc

NC = 2
NS = 16
L = 16
NW = NC * NS

B = 16384
D = 128
BPW = B // NW
CHUNKS = (64, 128, 128, 128, 64)
OFFS = (0, 64, 192, 320, 448)
SLOTS = (0, 64, 192, 320, 0)
NROWS = 448
RUNROLL = 4


def _body(uid_h, iid_h, ue_h, ie_h, ub_h, ib_h, gb_h, out_h,
          uidx_all, iidx_all, uev, iev, ubv, ibv, gbs, outv,
          sem0, sem1, sem2, sem3, sem4, semb):
    cid = lax.axis_index("c")
    sid = lax.axis_index("s")
    wid = sid * NC + cid
    base = wid * BPW
    sems = [sem0, sem1, sem2, sem3, sem4]
    lane = lax.iota(jnp.int32, L)

    pltpu.sync_copy(gb_h, gbs)

    pltpu.sync_copy(uid_h.at[pl.ds(base, BPW)], uidx_all)
    pltpu.sync_copy(iid_h.at[pl.ds(base, BPW)], iidx_all)

    def copies(c):
        isl = pl.ds(OFFS[c], CHUNKS[c])
        osl = pl.ds(SLOTS[c], CHUNKS[c])
        return (
            pltpu.make_async_copy(ue_h.at[uidx_all.at[isl]], uev.at[osl],
                                  sems[c]),
            pltpu.make_async_copy(ie_h.at[iidx_all.at[isl]], iev.at[osl],
                                  sems[c]),
        )

    for c in range(len(CHUNKS) - 1):
        for cp in copies(c):
            cp.start()
    bias_cps = [
        pltpu.async_copy(ub_h.at[uidx_all], ubv, semb),
        pltpu.async_copy(ib_h.at[iidx_all], ibv, semb),
    ]

    for c in range(len(CHUNKS)):
        for cp in copies(c):
            cp.wait()

        def group_body(g, carry2, c=c):
            def sub_body(rr, out16):
                for q in range(RUNROLL):
                    r = rr * RUNROLL + q
                    row = SLOTS[c] + g * L + r
                    pacc = (uev[row, pl.ds(0, L)] *
                            iev[row, pl.ds(0, L)])
                    for j in range(1, D // L):
                        pacc = pacc + (uev[row, pl.ds(j * L, L)] *
                                       iev[row, pl.ds(j * L, L)])
                    s = jnp.sum(pacc)
                    out16 = jnp.where(lane == r, s, out16)
                return out16

            out16 = lax.fori_loop(0, L // RUNROLL, sub_body,
                                  jnp.zeros((L,), jnp.float32))
            outv[pl.ds(OFFS[c] + g * L, L)] = out16
            return carry2

        lax.fori_loop(0, CHUNKS[c] // L, group_body, 0)

        if c == 0:
            for cp in copies(len(CHUNKS) - 1):
                cp.start()

    for cp in bias_cps:
        cp.wait()
    gb = gbs[...]

    def bias_body(g, carry):
        sl = pl.ds(g * L, L)
        outv[sl] = outv[sl] + ubv[sl] + ibv[sl] + gb
        return carry

    lax.fori_loop(0, BPW // L, bias_body, 0)

    pltpu.sync_copy(outv, out_h.at[pl.ds(base, BPW)])


@functools.partial(jax.jit, static_argnames=())
def kernel(user_ids, item_ids, user_emb, item_emb, user_bias, item_bias,
           global_bias):
    gb1 = jnp.broadcast_to(global_bias.astype(jnp.float32), (L,))
    mesh = plsc.VectorSubcoreMesh(core_axis_name="c", subcore_axis_name="s",
                                  num_cores=NC, num_subcores=NS)
    run = pl.kernel(
        _body,
        out_type=jax.ShapeDtypeStruct((B,), jnp.float32),
        mesh=mesh,
        compiler_params=pltpu.CompilerParams(needs_layout_passes=False),
        scratch_types=[
            pltpu.VMEM((BPW,), jnp.int32),
            pltpu.VMEM((BPW,), jnp.int32),
            pltpu.VMEM((NROWS, D), jnp.float32),
            pltpu.VMEM((NROWS, D), jnp.float32),
            pltpu.VMEM((BPW,), jnp.float32),
            pltpu.VMEM((BPW,), jnp.float32),
            pltpu.VMEM((L,), jnp.float32),
            pltpu.VMEM((BPW,), jnp.float32),
            pltpu.SemaphoreType.DMA,
            pltpu.SemaphoreType.DMA,
            pltpu.SemaphoreType.DMA,
            pltpu.SemaphoreType.DMA,
            pltpu.SemaphoreType.DMA,
            pltpu.SemaphoreType.DMA,
        ],
    )
    return run(user_ids.astype(jnp.int32), item_ids.astype(jnp.int32),
               user_emb, item_emb, user_bias, item_bias, gb1)


# speedup vs baseline: 1.0621x; 1.0621x over previous
import functools

import jax
import jax.numpy as jnp
from jax import lax
from jax.experimental import pallas as pl
from jax.experimental.pallas import tpu as pltpu
from jax.experimental.pallas import tpu_sc as plsc

NC = 2
NS = 16
L = 16
NW = NC * NS

B = 16384
D = 128
BPW = B // NW
CH = 128
CHUNKS = (64, 128, 128, 128, 64)
OFFS = (0, 64, 192, 320, 448)
NBUF = 2
RUNROLL = 4


def _body(uid_h, iid_h, ue_h, ie_h, ub_h, ib_h, gb_h, out_h,
          uidx_all, iidx_all, uev, iev, ubv, ibv, gbs, outv,
          sem0, sem1, semb):
    cid = lax.axis_index("c")
    sid = lax.axis_index("s")
    wid = sid * NC + cid
    base = wid * BPW
    sems = [sem0, sem1]
    lane = lax.iota(jnp.int32, L)

    pltpu.sync_copy(gb_h, gbs)

    pltpu.sync_copy(uid_h.at[pl.ds(base, BPW)], uidx_all)
    pltpu.sync_copy(iid_h.at[pl.ds(base, BPW)], iidx_all)

    def issue(c, b):
        isl = pl.ds(OFFS[c], CHUNKS[c])
        dsl = pl.ds(0, CHUNKS[c])
        pltpu.async_copy(ue_h.at[uidx_all.at[isl]], uev.at[b, dsl], sems[b])
        pltpu.async_copy(ie_h.at[iidx_all.at[isl]], iev.at[b, dsl], sems[b])

    def drain(c, b):
        isl = pl.ds(OFFS[c], CHUNKS[c])
        dsl = pl.ds(0, CHUNKS[c])
        pltpu.make_async_copy(ue_h.at[uidx_all.at[isl]], uev.at[b, dsl],
                              sems[b]).wait()
        pltpu.make_async_copy(ie_h.at[iidx_all.at[isl]], iev.at[b, dsl],
                              sems[b]).wait()

    issue(0, 0)
    issue(1, 1)

    bias_cps = []
    for q in range(BPW // 128):
        sl = pl.ds(q * 128, 128)
        bias_cps.append(pltpu.async_copy(ub_h.at[uidx_all.at[sl]],
                                         ubv.at[sl], semb))
        bias_cps.append(pltpu.async_copy(ib_h.at[iidx_all.at[sl]],
                                         ibv.at[sl], semb))

    for c in range(len(CHUNKS)):
        b = c % NBUF
        drain(c, b)

        def group_body(g, carry2, b=b, c=c):
            def sub_body(rr, out16):
                for q in range(RUNROLL):
                    r = rr * RUNROLL + q
                    row = g * L + r
                    pacc = (uev[b, row, pl.ds(0, L)] *
                            iev[b, row, pl.ds(0, L)])
                    for j in range(1, D // L):
                        pacc = pacc + (uev[b, row, pl.ds(j * L, L)] *
                                       iev[b, row, pl.ds(j * L, L)])
                    s = jnp.sum(pacc)
                    out16 = jnp.where(lane == r, s, out16)
                return out16

            out16 = lax.fori_loop(0, L // RUNROLL, sub_body,
                                  jnp.zeros((L,), jnp.float32))
            outv[pl.ds(OFFS[c] + g * L, L)] = out16
            return carry2

        lax.fori_loop(0, CHUNKS[c] // L, group_body, 0)

        if c + NBUF < len(CHUNKS):
            issue(c + NBUF, b)

    for cp in bias_cps:
        cp.wait()
    gb = gbs[...]

    def bias_body(g, carry):
        sl = pl.ds(g * L, L)
        outv[sl] = outv[sl] + ubv[sl] + ibv[sl] + gb
        return carry

    lax.fori_loop(0, BPW // L, bias_body, 0)

    pltpu.sync_copy(outv.at[pl.ds(0, BPW)], out_h.at[pl.ds(base, BPW)])


@functools.partial(jax.jit, static_argnames=())
def kernel(user_ids, item_ids, user_emb, item_emb, user_bias, item_bias,
           global_bias):
    gb1 = jnp.broadcast_to(global_bias.astype(jnp.float32), (L,))
    mesh = plsc.VectorSubcoreMesh(core_axis_name="c", subcore_axis_name="s",
                                  num_cores=NC, num_subcores=NS)
    run = pl.kernel(
        _body,
        out_type=jax.ShapeDtypeStruct((B,), jnp.float32),
        mesh=mesh,
        compiler_params=pltpu.CompilerParams(needs_layout_passes=False),
        scratch_types=[
            pltpu.VMEM((BPW,), jnp.int32),
            pltpu.VMEM((BPW,), jnp.int32),
            pltpu.VMEM((NBUF, CH, D), jnp.float32),
            pltpu.VMEM((NBUF, CH, D), jnp.float32),
            pltpu.VMEM((BPW,), jnp.float32),
            pltpu.VMEM((BPW,), jnp.float32),
            pltpu.VMEM((L,), jnp.float32),
            pltpu.VMEM((BPW + L,), jnp.float32),
            pltpu.SemaphoreType.DMA,
            pltpu.SemaphoreType.DMA,
            pltpu.SemaphoreType.DMA,
        ],
    )
    return run(user_ids.astype(jnp.int32), item_ids.astype(jnp.int32),
               user_emb, item_emb, user_bias, item_bias, gb1)
